# baseline (speedup 1.0000x reference)
import functools

import jax
import jax.numpy as jnp
from jax.experimental import pallas as pl
from jax.experimental.pallas import tpu as pltpu

_VMEM_LIMIT = 96 * 1024 * 1024
_BLK = 8


def _conv_stage(view, wk_ref, bk_ref, pool_scr, *, H, W, Cin, Cout, BLK):
    Ho, Wo = H // 2, W // 2
    patches = jnp.concatenate(
        [view(dh, dw) for dh in range(3) for dw in range(3)], axis=1)
    acc = jnp.dot(patches, wk_ref[...], preferred_element_type=jnp.float32)
    y = jnp.maximum(acc + bk_ref[...], 0.0)

    y4 = y.reshape(BLK * Ho, 2, W, Cout)
    rows = jnp.maximum(y4[:, 0], y4[:, 1])
    pool_scr[...] = rows.reshape(BLK * Ho * W, Cout)
    n = BLK * Ho * Wo
    return jnp.maximum(pool_scr[pl.ds(0, n, stride=2), :],
                       pool_scr[pl.ds(1, n, stride=2), :])


def _fused_kernel(x_ref, w1_ref, b1_ref, w2_ref, b2_ref, w3_ref, b3_ref,
                  wfc_ref, bfc_ref, o_ref,
                  x2_scr, x3_scr, ps1, ps2, ps3, *, BLK):
    p1 = _conv_stage(
        lambda dh, dw: x_ref[:, dh:dh + 64, dw:dw + 64, :].reshape(-1, 3),
        w1_ref, b1_ref, ps1, H=64, W=64, Cin=3, Cout=16, BLK=BLK)

    x2_scr[...] = jnp.zeros_like(x2_scr)
    x2_scr[:, 1:33, 1:33, :] = p1.reshape(BLK, 32, 32, 16).astype(jnp.bfloat16)

    p2 = _conv_stage(
        lambda dh, dw: x2_scr[:, dh:dh + 32, dw:dw + 32, :].reshape(-1, 16),
        w2_ref, b2_ref, ps2, H=32, W=32, Cin=16, Cout=32, BLK=BLK)

    x3_scr[...] = jnp.zeros_like(x3_scr)
    x3_scr[:, 1:17, 1:17, :] = p2.reshape(BLK, 16, 16, 32).astype(jnp.bfloat16)

    p3 = _conv_stage(
        lambda dh, dw: x3_scr[:, dh:dh + 16, dw:dw + 16, :].reshape(-1, 32),
        w3_ref, b3_ref, ps3, H=16, W=16, Cin=32, Cout=64, BLK=BLK)

    r = p3.reshape(BLK, 64, 64)
    l0 = jnp.sum(jnp.sum(r * wfc_ref[0], axis=2), axis=1, keepdims=True)
    l1 = jnp.sum(jnp.sum(r * wfc_ref[1], axis=2), axis=1, keepdims=True)
    lane = jax.lax.broadcasted_iota(jnp.int32, (BLK, 2), 1)
    o_ref[...] = jnp.where(lane == 0, l0, l1) + bfc_ref[...]


def _prep_conv_weights(w_oihw, b):
    Cout, Cin = w_oihw.shape[0], w_oihw.shape[1]
    wk = jnp.transpose(w_oihw, (2, 3, 1, 0)).reshape(9 * Cin, Cout)
    return wk.astype(jnp.bfloat16), b.reshape(1, Cout).astype(jnp.float32)


def kernel(x, w1, b1, w2, b2, w3, b3, wfc, bfc):
    B = x.shape[0]
    BLK = _BLK
    xp = jnp.pad(jnp.transpose(x, (0, 2, 3, 1)).astype(jnp.bfloat16),
                 ((0, 0), (1, 1), (1, 1), (0, 0)))

    wk1, bk1 = _prep_conv_weights(w1, b1)
    wk2, bk2 = _prep_conv_weights(w2, b2)
    wk3, bk3 = _prep_conv_weights(w3, b3)
    wfc_p = (wfc.reshape(64, 8, 8, 2).transpose(3, 1, 2, 0)
                .reshape(2, 64, 64).astype(jnp.float32))
    bfc_p = bfc.reshape(1, 2).astype(jnp.float32)

    kernel_fn = functools.partial(_fused_kernel, BLK=BLK)
    out = pl.pallas_call(
        kernel_fn,
        out_shape=jax.ShapeDtypeStruct((B, 2), jnp.float32),
        grid=(B // BLK,),
        in_specs=[
            pl.BlockSpec((BLK, 66, 66, 3), lambda i: (i, 0, 0, 0)),
            pl.BlockSpec((27, 16), lambda i: (0, 0)),
            pl.BlockSpec((1, 16), lambda i: (0, 0)),
            pl.BlockSpec((144, 32), lambda i: (0, 0)),
            pl.BlockSpec((1, 32), lambda i: (0, 0)),
            pl.BlockSpec((288, 64), lambda i: (0, 0)),
            pl.BlockSpec((1, 64), lambda i: (0, 0)),
            pl.BlockSpec((2, 64, 64), lambda i: (0, 0, 0)),
            pl.BlockSpec((1, 2), lambda i: (0, 0)),
        ],
        out_specs=pl.BlockSpec((BLK, 2), lambda i: (i, 0)),
        scratch_shapes=[
            pltpu.VMEM((BLK, 34, 34, 16), jnp.bfloat16),
            pltpu.VMEM((BLK, 18, 18, 32), jnp.bfloat16),
            pltpu.VMEM((BLK * 32 * 64, 16), jnp.float32),
            pltpu.VMEM((BLK * 16 * 32, 32), jnp.float32),
            pltpu.VMEM((BLK * 8 * 16, 64), jnp.float32),
        ],
        compiler_params=pltpu.CompilerParams(
            dimension_semantics=("parallel",),
            vmem_limit_bytes=_VMEM_LIMIT),
    )(xp, wk1, bk1, wk2, bk2, wk3, bk3, wfc_p, bfc_p)
    return out


# speedup vs baseline: 1.4792x; 1.4792x over previous
import functools

import jax
import jax.numpy as jnp
from jax.experimental import pallas as pl
from jax.experimental.pallas import tpu as pltpu

_VMEM_LIMIT = 96 * 1024 * 1024
_BLK = 8


def _conv_stage(view, wk_ref, bk_ref, pool_scr, *, H, W, Cin, Cout, BLK):
    Ho, Wo = H // 2, W // 2
    patches = jnp.concatenate(
        [view(dh, dw) for dh in range(3) for dw in range(3)], axis=1)
    acc = jnp.dot(patches, wk_ref[...], preferred_element_type=jnp.float32)
    y = jnp.maximum(acc + bk_ref[...], 0.0)

    y4 = y.reshape(BLK * Ho, 2, W, Cout)
    rows = jnp.maximum(y4[:, 0], y4[:, 1])
    pool_scr[...] = rows.reshape(BLK * Ho * W, Cout)
    n = BLK * Ho * Wo
    return jnp.maximum(pool_scr[pl.ds(0, n, stride=2), :],
                       pool_scr[pl.ds(1, n, stride=2), :])


def _fused_kernel(x_ref, w1_ref, b1_ref, w2_ref, b2_ref, w3_ref, b3_ref,
                  wfc_ref, bfc_ref, o_ref,
                  x1_scr, x2_scr, x3_scr, ps1, ps2, ps3, *, BLK):
    x1_scr[...] = jnp.zeros_like(x1_scr)
    x1_scr[:, 1:65, 1:65, :] = jnp.transpose(
        x_ref[...].astype(jnp.bfloat16), (0, 2, 3, 1))

    p1 = _conv_stage(
        lambda dh, dw: x1_scr[:, dh:dh + 64, dw:dw + 64, :].reshape(-1, 3),
        w1_ref, b1_ref, ps1, H=64, W=64, Cin=3, Cout=16, BLK=BLK)

    x2_scr[...] = jnp.zeros_like(x2_scr)
    x2_scr[:, 1:33, 1:33, :] = p1.reshape(BLK, 32, 32, 16).astype(jnp.bfloat16)

    p2 = _conv_stage(
        lambda dh, dw: x2_scr[:, dh:dh + 32, dw:dw + 32, :].reshape(-1, 16),
        w2_ref, b2_ref, ps2, H=32, W=32, Cin=16, Cout=32, BLK=BLK)

    x3_scr[...] = jnp.zeros_like(x3_scr)
    x3_scr[:, 1:17, 1:17, :] = p2.reshape(BLK, 16, 16, 32).astype(jnp.bfloat16)

    p3 = _conv_stage(
        lambda dh, dw: x3_scr[:, dh:dh + 16, dw:dw + 16, :].reshape(-1, 32),
        w3_ref, b3_ref, ps3, H=16, W=16, Cin=32, Cout=64, BLK=BLK)

    r = p3.reshape(BLK, 64, 64)
    l0 = jnp.sum(jnp.sum(r * wfc_ref[0], axis=2), axis=1, keepdims=True)
    l1 = jnp.sum(jnp.sum(r * wfc_ref[1], axis=2), axis=1, keepdims=True)
    lane = jax.lax.broadcasted_iota(jnp.int32, (BLK, 2), 1)
    o_ref[...] = jnp.where(lane == 0, l0, l1) + bfc_ref[...]


def _prep_conv_weights(w_oihw, b):
    Cout, Cin = w_oihw.shape[0], w_oihw.shape[1]
    wk = jnp.transpose(w_oihw, (2, 3, 1, 0)).reshape(9 * Cin, Cout)
    return wk.astype(jnp.bfloat16), b.reshape(1, Cout).astype(jnp.float32)


def kernel(x, w1, b1, w2, b2, w3, b3, wfc, bfc):
    B = x.shape[0]
    BLK = _BLK
    wk1, bk1 = _prep_conv_weights(w1, b1)
    wk2, bk2 = _prep_conv_weights(w2, b2)
    wk3, bk3 = _prep_conv_weights(w3, b3)
    wfc_p = (wfc.reshape(64, 8, 8, 2).transpose(3, 1, 2, 0)
                .reshape(2, 64, 64).astype(jnp.float32))
    bfc_p = bfc.reshape(1, 2).astype(jnp.float32)

    kernel_fn = functools.partial(_fused_kernel, BLK=BLK)
    out = pl.pallas_call(
        kernel_fn,
        out_shape=jax.ShapeDtypeStruct((B, 2), jnp.float32),
        grid=(B // BLK,),
        in_specs=[
            pl.BlockSpec((BLK, 3, 64, 64), lambda i: (i, 0, 0, 0)),
            pl.BlockSpec((27, 16), lambda i: (0, 0)),
            pl.BlockSpec((1, 16), lambda i: (0, 0)),
            pl.BlockSpec((144, 32), lambda i: (0, 0)),
            pl.BlockSpec((1, 32), lambda i: (0, 0)),
            pl.BlockSpec((288, 64), lambda i: (0, 0)),
            pl.BlockSpec((1, 64), lambda i: (0, 0)),
            pl.BlockSpec((2, 64, 64), lambda i: (0, 0, 0)),
            pl.BlockSpec((1, 2), lambda i: (0, 0)),
        ],
        out_specs=pl.BlockSpec((BLK, 2), lambda i: (i, 0)),
        scratch_shapes=[
            pltpu.VMEM((BLK, 66, 66, 3), jnp.bfloat16),
            pltpu.VMEM((BLK, 34, 34, 16), jnp.bfloat16),
            pltpu.VMEM((BLK, 18, 18, 32), jnp.bfloat16),
            pltpu.VMEM((BLK * 32 * 64, 16), jnp.float32),
            pltpu.VMEM((BLK * 16 * 32, 32), jnp.float32),
            pltpu.VMEM((BLK * 8 * 16, 64), jnp.float32),
        ],
        compiler_params=pltpu.CompilerParams(
            dimension_semantics=("parallel",),
            vmem_limit_bytes=_VMEM_LIMIT),
    )(x, wk1, bk1, wk2, bk2, wk3, bk3, wfc_p, bfc_p)
    return out


# speedup vs baseline: 10.4440x; 7.0606x over previous
import functools

import jax
import jax.numpy as jnp
import numpy as np
from jax.experimental import pallas as pl
from jax.experimental.pallas import tpu as pltpu

_VMEM_LIMIT = 100 * 1024 * 1024
_BLK = 8
_NEG = -1e30


def _stage1_band_indicator():
    dw = np.arange(3).reshape(3, 1, 1, 1)
    wi = np.arange(64).reshape(1, 64, 1, 1)
    p = np.arange(2).reshape(1, 1, 2, 1)
    u = np.arange(1, 33).reshape(1, 1, 1, 32)
    return (wi == 2 * u + p + dw - 3).astype(np.float32)


def _stageN_band_indicator(W):
    dw = np.arange(3).reshape(3, 1, 1, 1)
    ui = np.arange(W + 2).reshape(1, W + 2, 1, 1)
    p = np.arange(2).reshape(1, 1, 2, 1)
    u = np.arange(1, W // 2 + 1).reshape(1, 1, 1, W // 2)
    return (ui == 2 * u + p + dw - 2).astype(np.float32)


_I1 = _stage1_band_indicator()
_J2 = _stageN_band_indicator(32)
_J3 = _stageN_band_indicator(16)


def _band1(w1):
    r = jnp.einsum("oihd,dwpu->ihwpou", w1, _I1)
    r = jnp.pad(r, ((0, 0),) * 5 + ((1, 1),))
    return r.reshape(3 * 3 * 64, 2 * 16 * 34).astype(jnp.bfloat16)


def _bandN(w, ind, W):
    Cout, Cin = w.shape[0], w.shape[1]
    r = jnp.einsum("oihd,dzpu->hizpou", w, ind)
    r = jnp.pad(r, ((0, 0),) * 5 + ((1, 1),))
    return r.reshape(3 * Cin * (W + 2),
                     2 * Cout * (W // 2 + 2)).astype(jnp.bfloat16)


def _bias_ext(b, U):
    core = jnp.broadcast_to(b.reshape(1, -1, 1), (2, b.shape[0], U - 2))
    return jnp.pad(core, ((0, 0), (0, 0), (1, 1)),
                   constant_values=_NEG).reshape(1, -1).astype(jnp.float32)


def _conv_pool(x3_scr, wb_ref, be_ref, *, BLK, H, L, Nh):
    xc = jnp.concatenate(
        [x3_scr[:, dh:dh + H, :].reshape(BLK * H, L) for dh in range(3)],
        axis=1)
    acc = jnp.dot(xc, wb_ref[...], preferred_element_type=jnp.float32)
    y = jnp.maximum(acc + be_ref[...], 0.0)
    y2 = y.reshape(BLK * H // 2, 2, 2 * Nh)
    yr = jnp.maximum(y2[:, 0], y2[:, 1])
    return jnp.maximum(yr[:, :Nh], yr[:, Nh:])


def _fused_kernel(x_ref, wb1_ref, be1_ref, wb2_ref, be2_ref, wb3_ref,
                  be3_ref, wf_ref, bfc_ref, o_ref, xs1, xs2, xs3, *, BLK):
    xs1[...] = jnp.zeros_like(xs1)
    xs1[:, :, 1:65, :] = x_ref[...].astype(jnp.bfloat16)
    xc1 = jnp.concatenate(
        [xs1[:, ci, dh:dh + 64, :].reshape(BLK * 64, 64)
         for ci in range(3) for dh in range(3)], axis=1)
    acc = jnp.dot(xc1, wb1_ref[...], preferred_element_type=jnp.float32)
    y = jnp.maximum(acc + be1_ref[...], 0.0)
    y2 = y.reshape(BLK * 32, 2, 1088)
    yr = jnp.maximum(y2[:, 0], y2[:, 1])
    p1 = jnp.maximum(yr[:, :544], yr[:, 544:])

    xs2[...] = jnp.zeros_like(xs2)
    xs2[:, 1:33, :] = p1.reshape(BLK, 32, 544).astype(jnp.bfloat16)
    p2 = _conv_pool(xs2, wb2_ref, be2_ref, BLK=BLK, H=32, L=544, Nh=576)

    xs3[...] = jnp.zeros_like(xs3)
    xs3[:, 1:17, :] = p2.reshape(BLK, 16, 576).astype(jnp.bfloat16)
    p3 = _conv_pool(xs3, wb3_ref, be3_ref, BLK=BLK, H=16, L=576, Nh=640)

    r = p3.reshape(BLK, 8, 640)
    l0 = jnp.sum(jnp.sum(r * wf_ref[0], axis=2), axis=1, keepdims=True)
    l1 = jnp.sum(jnp.sum(r * wf_ref[1], axis=2), axis=1, keepdims=True)
    lane = jax.lax.broadcasted_iota(jnp.int32, (BLK, 2), 1)
    o_ref[...] = jnp.where(lane == 0, l0, l1) + bfc_ref[...]


def kernel(x, w1, b1, w2, b2, w3, b3, wfc, bfc):
    B = x.shape[0]
    BLK = _BLK
    wb1 = _band1(w1)
    wb2 = _bandN(w2, _J2, 32)
    wb3 = _bandN(w3, _J3, 16)
    be1 = _bias_ext(b1, 34)
    be2 = _bias_ext(b2, 18)
    be3 = _bias_ext(b3, 10)
    wf = jnp.pad(wfc.reshape(64, 8, 8, 2).transpose(3, 1, 0, 2),
                 ((0, 0), (0, 0), (0, 0), (1, 1))).reshape(2, 8, 640)
    wf = wf.astype(jnp.float32)
    bfc_p = bfc.reshape(1, 2).astype(jnp.float32)

    kernel_fn = functools.partial(_fused_kernel, BLK=BLK)
    out = pl.pallas_call(
        kernel_fn,
        out_shape=jax.ShapeDtypeStruct((B, 2), jnp.float32),
        grid=(B // BLK,),
        in_specs=[
            pl.BlockSpec((BLK, 3, 64, 64), lambda i: (i, 0, 0, 0)),
            pl.BlockSpec((576, 1088), lambda i: (0, 0)),
            pl.BlockSpec((1, 1088), lambda i: (0, 0)),
            pl.BlockSpec((1632, 1152), lambda i: (0, 0)),
            pl.BlockSpec((1, 1152), lambda i: (0, 0)),
            pl.BlockSpec((1728, 1280), lambda i: (0, 0)),
            pl.BlockSpec((1, 1280), lambda i: (0, 0)),
            pl.BlockSpec((2, 8, 640), lambda i: (0, 0, 0)),
            pl.BlockSpec((1, 2), lambda i: (0, 0)),
        ],
        out_specs=pl.BlockSpec((BLK, 2), lambda i: (i, 0)),
        scratch_shapes=[
            pltpu.VMEM((BLK, 3, 66, 64), jnp.bfloat16),
            pltpu.VMEM((BLK, 34, 544), jnp.bfloat16),
            pltpu.VMEM((BLK, 18, 576), jnp.bfloat16),
        ],
        compiler_params=pltpu.CompilerParams(
            dimension_semantics=("parallel",),
            vmem_limit_bytes=_VMEM_LIMIT),
    )(x, wb1, be1, wb2, be2, wb3, be3, wf, bfc_p)
    return out


# speedup vs baseline: 11.1092x; 1.0637x over previous
import functools

import jax
import jax.numpy as jnp
import numpy as np
from jax.experimental import pallas as pl
from jax.experimental.pallas import tpu as pltpu

_VMEM_LIMIT = 100 * 1024 * 1024
_BLK = 16
_NEG = -1e30


def _stage1_band_indicator():
    dw = np.arange(3).reshape(3, 1, 1, 1)
    wi = np.arange(64).reshape(1, 64, 1, 1)
    p = np.arange(2).reshape(1, 1, 2, 1)
    u = np.arange(1, 33).reshape(1, 1, 1, 32)
    return (wi == 2 * u + p + dw - 3).astype(np.float32)


def _stageN_band_indicator(W):
    dw = np.arange(3).reshape(3, 1, 1, 1)
    ui = np.arange(W + 2).reshape(1, W + 2, 1, 1)
    p = np.arange(2).reshape(1, 1, 2, 1)
    u = np.arange(1, W // 2 + 1).reshape(1, 1, 1, W // 2)
    return (ui == 2 * u + p + dw - 2).astype(np.float32)


_I1 = _stage1_band_indicator()
_J2 = _stageN_band_indicator(32)
_J3 = _stageN_band_indicator(16)


def _band1(w1):
    r = jnp.einsum("oihd,dwpu->ihwpou", w1, _I1)
    r = jnp.pad(r, ((0, 0),) * 5 + ((1, 1),))
    return r.reshape(3 * 3 * 64, 2 * 16 * 34).astype(jnp.bfloat16)


def _bandN(w, ind, W):
    Cout, Cin = w.shape[0], w.shape[1]
    r = jnp.einsum("oihd,dzpu->hizpou", w, ind)
    r = jnp.pad(r, ((0, 0),) * 5 + ((1, 1),))
    return r.reshape(3 * Cin * (W + 2),
                     2 * Cout * (W // 2 + 2)).astype(jnp.bfloat16)


def _bias_ext(b, U):
    core = jnp.broadcast_to(b.reshape(1, -1, 1), (2, b.shape[0], U - 2))
    return jnp.pad(core, ((0, 0), (0, 0), (1, 1)),
                   constant_values=_NEG).reshape(1, -1).astype(jnp.float32)


def _conv_pool(x3_scr, wb_ref, be_ref, *, BLK, H, L, Nh):
    xc = jnp.concatenate(
        [x3_scr[:, dh:dh + H, :].reshape(BLK * H, L) for dh in range(3)],
        axis=1)
    acc = jnp.dot(xc, wb_ref[...], preferred_element_type=jnp.float32)
    y = jnp.maximum(acc + be_ref[...], 0.0)
    y2 = y.reshape(BLK * H // 2, 2, 2 * Nh)
    yr = jnp.maximum(y2[:, 0], y2[:, 1])
    return jnp.maximum(yr[:, :Nh], yr[:, Nh:])


def _fused_kernel(x_ref, wb1_ref, be1_ref, wb2_ref, be2_ref, wb3_ref,
                  be3_ref, wf_ref, bfc_ref, o_ref, xs1, xs2, xs3, *, BLK):
    xs1[...] = jnp.zeros_like(xs1)
    xs1[:, :, 1:65, :] = x_ref[...].astype(jnp.bfloat16)
    xc1 = jnp.concatenate(
        [xs1[:, ci, dh:dh + 64, :].reshape(BLK * 64, 64)
         for ci in range(3) for dh in range(3)], axis=1)
    acc = jnp.dot(xc1, wb1_ref[...], preferred_element_type=jnp.float32)
    y = jnp.maximum(acc + be1_ref[...], 0.0)
    y2 = y.reshape(BLK * 32, 2, 1088)
    yr = jnp.maximum(y2[:, 0], y2[:, 1])
    p1 = jnp.maximum(yr[:, :544], yr[:, 544:])

    xs2[...] = jnp.zeros_like(xs2)
    xs2[:, 1:33, :] = p1.reshape(BLK, 32, 544).astype(jnp.bfloat16)
    p2 = _conv_pool(xs2, wb2_ref, be2_ref, BLK=BLK, H=32, L=544, Nh=576)

    xs3[...] = jnp.zeros_like(xs3)
    xs3[:, 1:17, :] = p2.reshape(BLK, 16, 576).astype(jnp.bfloat16)
    p3 = _conv_pool(xs3, wb3_ref, be3_ref, BLK=BLK, H=16, L=576, Nh=640)

    r = p3.reshape(BLK, 8, 640)
    l0 = jnp.sum(jnp.sum(r * wf_ref[0], axis=2), axis=1, keepdims=True)
    l1 = jnp.sum(jnp.sum(r * wf_ref[1], axis=2), axis=1, keepdims=True)
    lane = jax.lax.broadcasted_iota(jnp.int32, (BLK, 2), 1)
    o_ref[...] = jnp.where(lane == 0, l0, l1) + bfc_ref[...]


def kernel(x, w1, b1, w2, b2, w3, b3, wfc, bfc):
    B = x.shape[0]
    BLK = _BLK
    wb1 = _band1(w1)
    wb2 = _bandN(w2, _J2, 32)
    wb3 = _bandN(w3, _J3, 16)
    be1 = _bias_ext(b1, 34)
    be2 = _bias_ext(b2, 18)
    be3 = _bias_ext(b3, 10)
    wf = jnp.pad(wfc.reshape(64, 8, 8, 2).transpose(3, 1, 0, 2),
                 ((0, 0), (0, 0), (0, 0), (1, 1))).reshape(2, 8, 640)
    wf = wf.astype(jnp.float32)
    bfc_p = bfc.reshape(1, 2).astype(jnp.float32)

    kernel_fn = functools.partial(_fused_kernel, BLK=BLK)
    out = pl.pallas_call(
        kernel_fn,
        out_shape=jax.ShapeDtypeStruct((B, 2), jnp.float32),
        grid=(B // BLK,),
        in_specs=[
            pl.BlockSpec((BLK, 3, 64, 64), lambda i: (i, 0, 0, 0)),
            pl.BlockSpec((576, 1088), lambda i: (0, 0)),
            pl.BlockSpec((1, 1088), lambda i: (0, 0)),
            pl.BlockSpec((1632, 1152), lambda i: (0, 0)),
            pl.BlockSpec((1, 1152), lambda i: (0, 0)),
            pl.BlockSpec((1728, 1280), lambda i: (0, 0)),
            pl.BlockSpec((1, 1280), lambda i: (0, 0)),
            pl.BlockSpec((2, 8, 640), lambda i: (0, 0, 0)),
            pl.BlockSpec((1, 2), lambda i: (0, 0)),
        ],
        out_specs=pl.BlockSpec((BLK, 2), lambda i: (i, 0)),
        scratch_shapes=[
            pltpu.VMEM((BLK, 3, 66, 64), jnp.bfloat16),
            pltpu.VMEM((BLK, 34, 544), jnp.bfloat16),
            pltpu.VMEM((BLK, 18, 576), jnp.bfloat16),
        ],
        compiler_params=pltpu.CompilerParams(
            dimension_semantics=("parallel",),
            vmem_limit_bytes=_VMEM_LIMIT),
    )(x, wb1, be1, wb2, be2, wb3, be3, wf, bfc_p)
    return out


# speedup vs baseline: 11.1674x; 1.0052x over previous
import functools

import jax
import jax.numpy as jnp
import numpy as np
from jax.experimental import pallas as pl
from jax.experimental.pallas import tpu as pltpu

_VMEM_LIMIT = 100 * 1024 * 1024
_BLK = 32
_NEG = -1e30


def _stage1_band_indicator():
    dw = np.arange(3).reshape(3, 1, 1, 1)
    wi = np.arange(64).reshape(1, 64, 1, 1)
    p = np.arange(2).reshape(1, 1, 2, 1)
    u = np.arange(1, 33).reshape(1, 1, 1, 32)
    return (wi == 2 * u + p + dw - 3).astype(np.float32)


def _stageN_band_indicator(W):
    dw = np.arange(3).reshape(3, 1, 1, 1)
    ui = np.arange(W + 2).reshape(1, W + 2, 1, 1)
    p = np.arange(2).reshape(1, 1, 2, 1)
    u = np.arange(1, W // 2 + 1).reshape(1, 1, 1, W // 2)
    return (ui == 2 * u + p + dw - 2).astype(np.float32)


_I1 = _stage1_band_indicator()
_J2 = _stageN_band_indicator(32)
_J3 = _stageN_band_indicator(16)


def _band1(w1):
    r = jnp.einsum("oihd,dwpu->ihwpou", w1, _I1)
    r = jnp.pad(r, ((0, 0),) * 5 + ((1, 1),))
    return r.reshape(3 * 3 * 64, 2 * 16 * 34).astype(jnp.bfloat16)


def _bandN(w, ind, W):
    Cout, Cin = w.shape[0], w.shape[1]
    r = jnp.einsum("oihd,dzpu->hizpou", w, ind)
    r = jnp.pad(r, ((0, 0),) * 5 + ((1, 1),))
    return r.reshape(3 * Cin * (W + 2),
                     2 * Cout * (W // 2 + 2)).astype(jnp.bfloat16)


def _bias_ext(b, U):
    core = jnp.broadcast_to(b.reshape(1, -1, 1), (2, b.shape[0], U - 2))
    return jnp.pad(core, ((0, 0), (0, 0), (1, 1)),
                   constant_values=_NEG).reshape(1, -1).astype(jnp.float32)


def _conv_pool(x3_scr, wb_ref, be_ref, *, BLK, H, L, Nh):
    xc = jnp.concatenate(
        [x3_scr[:, dh:dh + H, :].reshape(BLK * H, L) for dh in range(3)],
        axis=1)
    acc = jnp.dot(xc, wb_ref[...], preferred_element_type=jnp.float32)
    y = jnp.maximum(acc + be_ref[...], 0.0)
    y2 = y.reshape(BLK * H // 2, 2, 2 * Nh)
    yr = jnp.maximum(y2[:, 0], y2[:, 1])
    return jnp.maximum(yr[:, :Nh], yr[:, Nh:])


def _fused_kernel(x_ref, wb1_ref, be1_ref, wb2_ref, be2_ref, wb3_ref,
                  be3_ref, wf_ref, bfc_ref, o_ref, xs1, xs2, xs3, *, BLK):
    xs1[...] = jnp.zeros_like(xs1)
    xs1[:, :, 1:65, :] = x_ref[...].astype(jnp.bfloat16)
    xc1 = jnp.concatenate(
        [xs1[:, ci, dh:dh + 64, :].reshape(BLK * 64, 64)
         for ci in range(3) for dh in range(3)], axis=1)
    acc = jnp.dot(xc1, wb1_ref[...], preferred_element_type=jnp.float32)
    y = jnp.maximum(acc + be1_ref[...], 0.0)
    y2 = y.reshape(BLK * 32, 2, 1088)
    yr = jnp.maximum(y2[:, 0], y2[:, 1])
    p1 = jnp.maximum(yr[:, :544], yr[:, 544:])

    xs2[...] = jnp.zeros_like(xs2)
    xs2[:, 1:33, :] = p1.reshape(BLK, 32, 544).astype(jnp.bfloat16)
    p2 = _conv_pool(xs2, wb2_ref, be2_ref, BLK=BLK, H=32, L=544, Nh=576)

    xs3[...] = jnp.zeros_like(xs3)
    xs3[:, 1:17, :] = p2.reshape(BLK, 16, 576).astype(jnp.bfloat16)
    p3 = _conv_pool(xs3, wb3_ref, be3_ref, BLK=BLK, H=16, L=576, Nh=640)

    r = p3.reshape(BLK, 8, 640)
    l0 = jnp.sum(jnp.sum(r * wf_ref[0], axis=2), axis=1, keepdims=True)
    l1 = jnp.sum(jnp.sum(r * wf_ref[1], axis=2), axis=1, keepdims=True)
    lane = jax.lax.broadcasted_iota(jnp.int32, (BLK, 2), 1)
    o_ref[...] = jnp.where(lane == 0, l0, l1) + bfc_ref[...]


def kernel(x, w1, b1, w2, b2, w3, b3, wfc, bfc):
    B = x.shape[0]
    BLK = _BLK
    wb1 = _band1(w1)
    wb2 = _bandN(w2, _J2, 32)
    wb3 = _bandN(w3, _J3, 16)
    be1 = _bias_ext(b1, 34)
    be2 = _bias_ext(b2, 18)
    be3 = _bias_ext(b3, 10)
    wf = jnp.pad(wfc.reshape(64, 8, 8, 2).transpose(3, 1, 0, 2),
                 ((0, 0), (0, 0), (0, 0), (1, 1))).reshape(2, 8, 640)
    wf = wf.astype(jnp.float32)
    bfc_p = bfc.reshape(1, 2).astype(jnp.float32)

    kernel_fn = functools.partial(_fused_kernel, BLK=BLK)
    out = pl.pallas_call(
        kernel_fn,
        out_shape=jax.ShapeDtypeStruct((B, 2), jnp.float32),
        grid=(B // BLK,),
        in_specs=[
            pl.BlockSpec((BLK, 3, 64, 64), lambda i: (i, 0, 0, 0)),
            pl.BlockSpec((576, 1088), lambda i: (0, 0)),
            pl.BlockSpec((1, 1088), lambda i: (0, 0)),
            pl.BlockSpec((1632, 1152), lambda i: (0, 0)),
            pl.BlockSpec((1, 1152), lambda i: (0, 0)),
            pl.BlockSpec((1728, 1280), lambda i: (0, 0)),
            pl.BlockSpec((1, 1280), lambda i: (0, 0)),
            pl.BlockSpec((2, 8, 640), lambda i: (0, 0, 0)),
            pl.BlockSpec((1, 2), lambda i: (0, 0)),
        ],
        out_specs=pl.BlockSpec((BLK, 2), lambda i: (i, 0)),
        scratch_shapes=[
            pltpu.VMEM((BLK, 3, 66, 64), jnp.bfloat16),
            pltpu.VMEM((BLK, 34, 544), jnp.bfloat16),
            pltpu.VMEM((BLK, 18, 576), jnp.bfloat16),
        ],
        compiler_params=pltpu.CompilerParams(
            dimension_semantics=("parallel",),
            vmem_limit_bytes=_VMEM_LIMIT),
    )(x, wb1, be1, wb2, be2, wb3, be3, wf, bfc_p)
    return out
